# baseline (speedup 1.0000x reference)
import numpy as np

import jax
import jax.numpy as jnp
from jax.experimental import pallas as pl
from jax.experimental.pallas import tpu as pltpu

_B, _T, _H, _W, _D = 8, 64, 16, 16, 256
_NSAMP = _B * _T
_NUM_FREQS = _T // 2
_NUM_SEG = 14
_TOP_K = 3
_BS = 32
_GRID = _NSAMP // _BS

_t = np.arange(_T)[:, None]
_k = np.arange(1, _NUM_FREQS + 1)[None, :]
_ang = 2.0 * np.pi * _t * _k / _T
_DFT_COS = (np.cos(_ang) / np.sqrt(_T)).astype(np.float32)
_DFT_SIN = (-np.sin(_ang) / np.sqrt(_T)).astype(np.float32)


def _tower_body(v, w_ref, b_ref, fw_ref, fb_ref):
    h = _H
    m = v.shape[0]
    v = v.reshape(m, h, h // 2, 2 * _D)
    for i in range(4):
        vp = v.reshape(m, h // 2, 2, h // 2, 2 * _D)
        mm = m * (h // 2) * (h // 2)
        ve = vp[:, :, 0].reshape(mm, 2 * _D)
        vo = vp[:, :, 1].reshape(mm, 2 * _D)
        t = (jnp.dot(ve, w_ref[i, 0], preferred_element_type=jnp.float32)
             + jnp.dot(vo, w_ref[i, 1], preferred_element_type=jnp.float32))
        t = t + b_ref[i]
        t = jnp.maximum(t, 0.2 * t)
        h = h // 2
        if h > 1:
            v = t.reshape(m, h, h // 2, 2 * _D)
    return jnp.dot(t, fw_ref[...], preferred_element_type=jnp.float32) + fb_ref[0, 0]


def _tower_kernel(x_ref, w_ref, b_ref, fw_ref, fb_ref, y_ref):
    y_ref[...] = _tower_body(x_ref[...], w_ref, b_ref, fw_ref, fb_ref)


def _gate_kernel(y_ref, cos_ref, sin_ref, wg_ref, out_ref):
    y = y_ref[...]
    re = jnp.dot(y, cos_ref[...], preferred_element_type=jnp.float32)
    im = jnp.dot(y, sin_ref[...], preferred_element_type=jnp.float32)
    amp = jnp.sqrt(re * re + im * im)
    logits = jnp.dot(amp, wg_ref[...], preferred_element_type=jnp.float32)
    cols = jax.lax.broadcasted_iota(jnp.int32, (_B, _NUM_SEG), 1)
    work = logits
    vals, idxs = [], []
    for _ in range(_TOP_K):
        mx = jnp.max(work, axis=-1, keepdims=True)
        eq = work == mx
        idx = jnp.min(jnp.where(eq, cols, _NUM_SEG), axis=-1, keepdims=True)
        vals.append(mx)
        idxs.append(idx)
        work = jnp.where(cols == idx, -1e30, work)
    es = [jnp.exp(v - vals[0]) for v in vals]
    denom = es[0] + es[1] + es[2]
    gates = jnp.zeros((_B, _NUM_SEG), dtype=jnp.float32)
    for e, idx in zip(es, idxs):
        gates = gates + jnp.where(cols == idx, e / denom, 0.0)
    out_ref[...] = gates


def kernel(x, training, conv_w, conv_b, bn_gamma, bn_beta, bn_mean, bn_var,
           fuse_w, fuse_b, w_gate, w_noise):
    del training, w_noise
    x4 = x.reshape(_NSAMP, _H, _W, _D)
    scale = bn_gamma * jax.lax.rsqrt(bn_var + 1e-5)
    wt = conv_w.transpose(0, 3, 4, 2, 1).reshape(4, 2, 2 * _D, _D)
    wt = wt * scale[:, None, None, :]
    beff = (conv_b - bn_mean) * scale + bn_beta
    fb = fuse_b.reshape(1, 1)

    y = pl.pallas_call(
        _tower_kernel,
        grid=(_GRID,),
        in_specs=[
            pl.BlockSpec((_BS, _H, _W, _D), lambda i: (i, 0, 0, 0)),
            pl.BlockSpec((4, 2, 2 * _D, _D), lambda i: (0, 0, 0, 0)),
            pl.BlockSpec((4, _D), lambda i: (0, 0)),
            pl.BlockSpec((_D, 1), lambda i: (0, 0)),
            pl.BlockSpec((1, 1), lambda i: (0, 0)),
        ],
        out_specs=pl.BlockSpec((_BS, 1), lambda i: (i, 0)),
        out_shape=jax.ShapeDtypeStruct((_NSAMP, 1), jnp.float32),
        compiler_params=pltpu.CompilerParams(
            dimension_semantics=("parallel",)),
    )(x4, wt, beff, fuse_w, fb)

    y2 = y.reshape(_B, _T)
    gates = pl.pallas_call(
        _gate_kernel,
        in_specs=[
            pl.BlockSpec((_B, _T), lambda: (0, 0)),
            pl.BlockSpec((_T, _NUM_FREQS), lambda: (0, 0)),
            pl.BlockSpec((_T, _NUM_FREQS), lambda: (0, 0)),
            pl.BlockSpec((_NUM_FREQS, _NUM_SEG), lambda: (0, 0)),
        ],
        out_specs=pl.BlockSpec((_B, _NUM_SEG), lambda: (0, 0)),
        out_shape=jax.ShapeDtypeStruct((_B, _NUM_SEG), jnp.float32),
    )(y2, jnp.asarray(_DFT_COS), jnp.asarray(_DFT_SIN), w_gate)
    return gates


# speedup vs baseline: 1.0702x; 1.0702x over previous
import numpy as np

import jax
import jax.numpy as jnp
from jax.experimental import pallas as pl
from jax.experimental.pallas import tpu as pltpu

_B, _T, _H, _W, _D = 8, 64, 16, 16, 256
_NSAMP = _B * _T
_NUM_FREQS = _T // 2
_NUM_SEG = 14
_TOP_K = 3
_BS = 64
_GRID = _NSAMP // _BS

_t = np.arange(_T)[:, None]
_k = np.arange(1, _NUM_FREQS + 1)[None, :]
_ang = 2.0 * np.pi * _t * _k / _T
_DFT_COS = (np.cos(_ang) / np.sqrt(_T)).astype(np.float32)
_DFT_SIN = (-np.sin(_ang) / np.sqrt(_T)).astype(np.float32)


def _tower_body(v, w_ref, b_ref, fw_ref, fb_ref):
    h = _H
    m = v.shape[0]
    v = v.reshape(m, h, h // 2, 2 * _D)
    for i in range(4):
        vp = v.reshape(m, h // 2, 2, h // 2, 2 * _D)
        mm = m * (h // 2) * (h // 2)
        ve = vp[:, :, 0].reshape(mm, 2 * _D)
        vo = vp[:, :, 1].reshape(mm, 2 * _D)
        t = (jnp.dot(ve, w_ref[i, 0], preferred_element_type=jnp.float32)
             + jnp.dot(vo, w_ref[i, 1], preferred_element_type=jnp.float32))
        t = t + b_ref[i]
        t = jnp.maximum(t, 0.2 * t)
        h = h // 2
        if h > 1:
            v = t.reshape(m, h, h // 2, 2 * _D)
    return jnp.dot(t, fw_ref[...], preferred_element_type=jnp.float32) + fb_ref[0, 0]


def _tower_kernel(x_ref, w_ref, b_ref, fw_ref, fb_ref, y_ref):
    y_ref[...] = _tower_body(x_ref[...], w_ref, b_ref, fw_ref, fb_ref)


def _gate_kernel(y_ref, cos_ref, sin_ref, wg_ref, out_ref):
    y = y_ref[...]
    re = jnp.dot(y, cos_ref[...], preferred_element_type=jnp.float32)
    im = jnp.dot(y, sin_ref[...], preferred_element_type=jnp.float32)
    amp = jnp.sqrt(re * re + im * im)
    logits = jnp.dot(amp, wg_ref[...], preferred_element_type=jnp.float32)
    cols = jax.lax.broadcasted_iota(jnp.int32, (_B, _NUM_SEG), 1)
    work = logits
    vals, idxs = [], []
    for _ in range(_TOP_K):
        mx = jnp.max(work, axis=-1, keepdims=True)
        eq = work == mx
        idx = jnp.min(jnp.where(eq, cols, _NUM_SEG), axis=-1, keepdims=True)
        vals.append(mx)
        idxs.append(idx)
        work = jnp.where(cols == idx, -1e30, work)
    es = [jnp.exp(v - vals[0]) for v in vals]
    denom = es[0] + es[1] + es[2]
    gates = jnp.zeros((_B, _NUM_SEG), dtype=jnp.float32)
    for e, idx in zip(es, idxs):
        gates = gates + jnp.where(cols == idx, e / denom, 0.0)
    out_ref[...] = gates


def kernel(x, training, conv_w, conv_b, bn_gamma, bn_beta, bn_mean, bn_var,
           fuse_w, fuse_b, w_gate, w_noise):
    del training, w_noise
    x4 = x.reshape(_NSAMP, _H, _W, _D)
    scale = bn_gamma * jax.lax.rsqrt(bn_var + 1e-5)
    wt = conv_w.transpose(0, 3, 4, 2, 1).reshape(4, 2, 2 * _D, _D)
    wt = wt * scale[:, None, None, :]
    beff = (conv_b - bn_mean) * scale + bn_beta
    fb = fuse_b.reshape(1, 1)

    y = pl.pallas_call(
        _tower_kernel,
        grid=(_GRID,),
        in_specs=[
            pl.BlockSpec((_BS, _H, _W, _D), lambda i: (i, 0, 0, 0)),
            pl.BlockSpec((4, 2, 2 * _D, _D), lambda i: (0, 0, 0, 0)),
            pl.BlockSpec((4, _D), lambda i: (0, 0)),
            pl.BlockSpec((_D, 1), lambda i: (0, 0)),
            pl.BlockSpec((1, 1), lambda i: (0, 0)),
        ],
        out_specs=pl.BlockSpec((_BS, 1), lambda i: (i, 0)),
        out_shape=jax.ShapeDtypeStruct((_NSAMP, 1), jnp.float32),
        compiler_params=pltpu.CompilerParams(
            dimension_semantics=("parallel",)),
    )(x4, wt, beff, fuse_w, fb)

    y2 = y.reshape(_B, _T)
    gates = pl.pallas_call(
        _gate_kernel,
        in_specs=[
            pl.BlockSpec((_B, _T), lambda: (0, 0)),
            pl.BlockSpec((_T, _NUM_FREQS), lambda: (0, 0)),
            pl.BlockSpec((_T, _NUM_FREQS), lambda: (0, 0)),
            pl.BlockSpec((_NUM_FREQS, _NUM_SEG), lambda: (0, 0)),
        ],
        out_specs=pl.BlockSpec((_B, _NUM_SEG), lambda: (0, 0)),
        out_shape=jax.ShapeDtypeStruct((_B, _NUM_SEG), jnp.float32),
    )(y2, jnp.asarray(_DFT_COS), jnp.asarray(_DFT_SIN), w_gate)
    return gates


# speedup vs baseline: 1.1159x; 1.0427x over previous
import numpy as np

import jax
import jax.numpy as jnp
from jax.experimental import pallas as pl
from jax.experimental.pallas import tpu as pltpu

_B, _T, _H, _W, _D = 8, 64, 16, 16, 256
_NSAMP = _B * _T
_NUM_FREQS = _T // 2
_NUM_SEG = 14
_TOP_K = 3
_BS = 64
_GRID = _NSAMP // _BS

_t = np.arange(_T)[:, None]
_k = np.arange(1, _NUM_FREQS + 1)[None, :]
_ang = 2.0 * np.pi * _t * _k / _T
_DFT_COS = (np.cos(_ang) / np.sqrt(_T)).astype(np.float32)
_DFT_SIN = (-np.sin(_ang) / np.sqrt(_T)).astype(np.float32)


def _tower_body(v, w_ref, b_ref, fw_ref, fb_ref):
    h = _H
    m = v.shape[0]
    v = v.reshape(m, h, h // 2, 2 * _D)
    for i in range(4):
        vp = v.reshape(m, h // 2, 2, h // 2, 2 * _D)
        mm = m * (h // 2) * (h // 2)
        ve = vp[:, :, 0].reshape(mm, 2 * _D)
        vo = vp[:, :, 1].reshape(mm, 2 * _D)
        t = (jnp.dot(ve, w_ref[i, 0], preferred_element_type=jnp.float32)
             + jnp.dot(vo, w_ref[i, 1], preferred_element_type=jnp.float32))
        t = t + b_ref[i]
        t = jnp.maximum(t, 0.2 * t)
        h = h // 2
        if h > 1:
            v = t.reshape(m, h, h // 2, 2 * _D)
    return jnp.dot(t, fw_ref[...], preferred_element_type=jnp.float32) + fb_ref[0, 0]


def _gate_math(y, cos_ref, sin_ref, wg_ref):
    re = jnp.dot(y, cos_ref[...], preferred_element_type=jnp.float32)
    im = jnp.dot(y, sin_ref[...], preferred_element_type=jnp.float32)
    amp = jnp.sqrt(re * re + im * im)
    logits = jnp.dot(amp, wg_ref[...], preferred_element_type=jnp.float32)
    cols = jax.lax.broadcasted_iota(jnp.int32, (_B, _NUM_SEG), 1)
    work = logits
    vals, idxs = [], []
    for _ in range(_TOP_K):
        mx = jnp.max(work, axis=-1, keepdims=True)
        eq = work == mx
        idx = jnp.min(jnp.where(eq, cols, _NUM_SEG), axis=-1, keepdims=True)
        vals.append(mx)
        idxs.append(idx)
        work = jnp.where(cols == idx, -1e30, work)
    es = [jnp.exp(v - vals[0]) for v in vals]
    denom = es[0] + es[1] + es[2]
    gates = jnp.zeros((_B, _NUM_SEG), dtype=jnp.float32)
    for e, idx in zip(es, idxs):
        gates = gates + jnp.where(cols == idx, e / denom, 0.0)
    return gates


def _fused_kernel(x_ref, w_ref, b_ref, fw_ref, fb_ref, cos_ref, sin_ref,
                  wg_ref, out_ref, ysc_ref):
    i = pl.program_id(0)
    yblk = _tower_body(x_ref[...], w_ref, b_ref, fw_ref, fb_ref)
    ysc_ref[pl.ds(i, 1), :] = yblk.reshape(1, _T)

    @pl.when(i == _GRID - 1)
    def _():
        out_ref[...] = _gate_math(ysc_ref[...], cos_ref, sin_ref, wg_ref)


def kernel(x, training, conv_w, conv_b, bn_gamma, bn_beta, bn_mean, bn_var,
           fuse_w, fuse_b, w_gate, w_noise):
    del training, w_noise
    x4 = x.reshape(_NSAMP, _H, _W, _D)
    scale = bn_gamma * jax.lax.rsqrt(bn_var + 1e-5)
    wt = conv_w.transpose(0, 3, 4, 2, 1).reshape(4, 2, 2 * _D, _D)
    wt = wt * scale[:, None, None, :]
    beff = (conv_b - bn_mean) * scale + bn_beta
    fb = fuse_b.reshape(1, 1)

    gates = pl.pallas_call(
        _fused_kernel,
        grid=(_GRID,),
        in_specs=[
            pl.BlockSpec((_BS, _H, _W, _D), lambda i: (i, 0, 0, 0)),
            pl.BlockSpec((4, 2, 2 * _D, _D), lambda i: (0, 0, 0, 0)),
            pl.BlockSpec((4, _D), lambda i: (0, 0)),
            pl.BlockSpec((_D, 1), lambda i: (0, 0)),
            pl.BlockSpec((1, 1), lambda i: (0, 0)),
            pl.BlockSpec((_T, _NUM_FREQS), lambda i: (0, 0)),
            pl.BlockSpec((_T, _NUM_FREQS), lambda i: (0, 0)),
            pl.BlockSpec((_NUM_FREQS, _NUM_SEG), lambda i: (0, 0)),
        ],
        out_specs=pl.BlockSpec((_B, _NUM_SEG), lambda i: (0, 0)),
        out_shape=jax.ShapeDtypeStruct((_B, _NUM_SEG), jnp.float32),
        scratch_shapes=[pltpu.VMEM((_B, _T), jnp.float32)],
        compiler_params=pltpu.CompilerParams(
            dimension_semantics=("arbitrary",)),
    )(x4, wt, beff, fuse_w, fb,
      jnp.asarray(_DFT_COS), jnp.asarray(_DFT_SIN), w_gate)
    return gates
